# Initial kernel scaffold; baseline (speedup 1.0000x reference)
import jax
import jax.numpy as jnp
from jax.experimental import pallas as pl


def kernel(points, cluster_ids, common_cluster_ids, spatial_features, spatial_features_2d, spatial_features_stride, batch_size):
    raise NotImplementedError("write your pallas kernel here")


# speedup vs baseline: 10.4521x; 10.4521x over previous
import jax
import jax.numpy as jnp
from jax import lax
from jax.experimental import pallas as pl
from jax.experimental.pallas import tpu as pltpu
from jax.experimental.pallas import tpu_sc as plsc

NKP = 1024
NCLU = 10
H = 188
W = 188
PCR_X0 = 0.0
PCR_Y0 = -40.0
VOX_X = 0.05
VOX_Y = 0.05
SUB_R = 8
SUB_C = 2048
KR = 8
KC = 128


def _fps_body(stride_ref, x_ref, y_ref, z_ref, lab_ref, mi_ref, mf_ref):
    x = x_ref[0]
    y = y_ref[0]
    z = z_ref[0]
    lab = lab_ref[0]
    npts = SUB_R * SUB_C
    pos = (lax.broadcasted_iota(jnp.int32, (SUB_R, SUB_C), 0) * SUB_C
           + lax.broadcasted_iota(jnp.int32, (SUB_R, SUB_C), 1))
    kpos = (lax.broadcasted_iota(jnp.int32, (KR, KC), 0) * KC
            + lax.broadcasted_iota(jnp.int32, (KR, KC), 1))

    def step(i, carry):
        dists, sel, kx, ky, klab = carry
        m = pos == sel
        qx = jnp.sum(jnp.where(m, x, 0.0))
        qy = jnp.sum(jnp.where(m, y, 0.0))
        qz = jnp.sum(jnp.where(m, z, 0.0))
        ql = jnp.sum(jnp.where(m, lab, 0))
        here = kpos == i
        kx = jnp.where(here, qx, kx)
        ky = jnp.where(here, qy, ky)
        klab = jnp.where(here, ql, klab)
        dx = x - qx
        dy = y - qy
        dz = z - qz
        d = dx * dx + dy * dy + dz * dz
        dists = jnp.minimum(dists, d)
        mx = jnp.max(dists)
        cand = jnp.where(dists == mx, pos, jnp.int32(npts))
        sel = jnp.min(cand)
        return dists, sel, kx, ky, klab

    dists0 = jnp.full((SUB_R, SUB_C), jnp.inf, jnp.float32)
    zf = jnp.zeros((KR, KC), jnp.float32)
    zi = jnp.zeros((KR, KC), jnp.int32)
    _, _, kx, ky, klab = lax.fori_loop(
        0, NKP, step, (dists0, jnp.int32(0), zf, zf, zi))

    stride_f = stride_ref[0]
    xi = (kx - PCR_X0) / jnp.float32(VOX_X) / stride_f
    yi = (ky - PCR_Y0) / jnp.float32(VOX_Y) / stride_f
    x0 = jnp.floor(xi).astype(jnp.int32)
    y0 = jnp.floor(yi).astype(jnp.int32)
    x1 = x0 + 1
    y1 = y0 + 1
    x0 = jnp.clip(x0, 0, W - 1)
    x1 = jnp.clip(x1, 0, W - 1)
    y0 = jnp.clip(y0, 0, H - 1)
    y1 = jnp.clip(y1, 0, H - 1)
    x0f = x0.astype(jnp.float32)
    x1f = x1.astype(jnp.float32)
    y0f = y0.astype(jnp.float32)
    y1f = y1.astype(jnp.float32)
    mi_ref[0, 0] = y0 * W + x0
    mi_ref[0, 1] = y1 * W + x0
    mi_ref[0, 2] = y0 * W + x1
    mi_ref[0, 3] = y1 * W + x1
    mi_ref[0, 4] = klab
    mf_ref[0, 0] = (x1f - xi) * (y1f - yi)
    mf_ref[0, 1] = (x1f - xi) * (yi - y0f)
    mf_ref[0, 2] = (xi - x0f) * (y1f - yi)
    mf_ref[0, 3] = (xi - x0f) * (yi - y0f)


def _fps_call(stride_arr, xs, ys, zs, labs):
    b = xs.shape[0]
    return pl.pallas_call(
        _fps_body,
        grid=(b,),
        in_specs=[
            pl.BlockSpec(memory_space=pltpu.SMEM),
            pl.BlockSpec((1, SUB_R, SUB_C), lambda i: (i, 0, 0)),
            pl.BlockSpec((1, SUB_R, SUB_C), lambda i: (i, 0, 0)),
            pl.BlockSpec((1, SUB_R, SUB_C), lambda i: (i, 0, 0)),
            pl.BlockSpec((1, SUB_R, SUB_C), lambda i: (i, 0, 0)),
        ],
        out_specs=[
            pl.BlockSpec((1, 5, KR, KC), lambda i: (i, 0, 0, 0)),
            pl.BlockSpec((1, 4, KR, KC), lambda i: (i, 0, 0, 0)),
        ],
        out_shape=[
            jax.ShapeDtypeStruct((b, 5, KR, KC), jnp.int32),
            jax.ShapeDtypeStruct((b, 4, KR, KC), jnp.float32),
        ],
    )(stride_arr, xs, ys, zs, labs)


def _sc_body(sf1, sf2, mi, mf, out, img_v, idx_v, wts_v, oacc_v):
    cid = lax.axis_index("c")
    sid = lax.axis_index("s")
    neg_inf = jnp.float32(-jnp.inf)
    n_grp = NKP // 16
    hw = H * W
    cout = 512
    lane0 = lax.broadcasted_iota(jnp.int32, (16,), 0) == 0

    def run(tbl, chan_off):
        b = sid // 4
        c0 = (sid % 4) * 64
        row0 = b * 256 + c0
        for q in range(5):
            pltpu.sync_copy(mi.at[pl.ds((b * 5 + q) * NKP, NKP)],
                            idx_v.at[pl.ds(q * NKP, NKP)])
        for q in range(4):
            pltpu.sync_copy(mf.at[pl.ds((b * 4 + q) * NKP, NKP)],
                            wts_v.at[pl.ds(q * NKP, NKP)])

        def chan_body(cc, carry):
            pltpu.sync_copy(tbl.at[pl.ds((row0 + cc) * hw, hw)], img_v)

            def grp(g, accs):
                st = pl.multiple_of(g * 16, 16)
                ia = idx_v[pl.ds(0 * NKP + st, 16)]
                ib = idx_v[pl.ds(1 * NKP + st, 16)]
                ic = idx_v[pl.ds(2 * NKP + st, 16)]
                idd = idx_v[pl.ds(3 * NKP + st, 16)]
                lbl = idx_v[pl.ds(4 * NKP + st, 16)]
                va = plsc.load_gather(img_v, [ia])
                vb = plsc.load_gather(img_v, [ib])
                vc = plsc.load_gather(img_v, [ic])
                vd = plsc.load_gather(img_v, [idd])
                wa = wts_v[pl.ds(0 * NKP + st, 16)]
                wb = wts_v[pl.ds(1 * NKP + st, 16)]
                wc = wts_v[pl.ds(2 * NKP + st, 16)]
                wd = wts_v[pl.ds(3 * NKP + st, 16)]
                feat = va * wa + vb * wb + vc * wc + vd * wd
                return tuple(
                    jnp.maximum(accs[j], jnp.where(lbl == j, feat, neg_inf))
                    for j in range(NCLU))

            accs0 = tuple(
                jnp.full((16,), neg_inf, jnp.float32) for _ in range(NCLU))
            accs = lax.fori_loop(0, n_grp, grp, accs0)
            ccv = jnp.broadcast_to(cc, (16,)).astype(jnp.int32)
            for j in range(NCLU):
                vm = jnp.broadcast_to(jnp.max(accs[j]), (16,))
                plsc.store_scatter(oacc_v, [ccv + j * 64], vm, mask=lane0)
            return carry

        lax.fori_loop(0, 64, chan_body, 0)
        for j in range(NCLU):
            pltpu.sync_copy(
                oacc_v.at[pl.ds(j * 64, 64)],
                out.at[pl.ds((b * NCLU + j) * cout + chan_off + c0, 64)])

    @pl.when(cid == 0)
    def _():
        run(sf1, 0)

    @pl.when(cid == 1)
    def _():
        run(sf2, 256)


def _sc_call(sf1r, sf2r, mi_r, mf_r, nb, cout):
    mesh = plsc.VectorSubcoreMesh(core_axis_name="c", subcore_axis_name="s")
    return pl.kernel(
        _sc_body,
        out_type=jax.ShapeDtypeStruct((nb * NCLU * cout,), jnp.float32),
        mesh=mesh,
        compiler_params=pltpu.CompilerParams(needs_layout_passes=False),
        scratch_types=[
            pltpu.VMEM((H * W,), jnp.float32),
            pltpu.VMEM((5 * NKP,), jnp.int32),
            pltpu.VMEM((4 * NKP,), jnp.float32),
            pltpu.VMEM((NCLU * 64,), jnp.float32),
        ],
    )(sf1r, sf2r, mi_r, mf_r)


def kernel(points, cluster_ids, common_cluster_ids, spatial_features,
           spatial_features_2d, spatial_features_stride, batch_size):
    nb, p = cluster_ids.shape
    pts = points.reshape(nb, p, 4)
    bs_zero = jnp.asarray(batch_size).astype(points.dtype) - nb
    xs = (pts[:, :, 1] + bs_zero).reshape(nb, SUB_R, SUB_C)
    ys = (pts[:, :, 2] + bs_zero).reshape(nb, SUB_R, SUB_C)
    zs = (pts[:, :, 3] + bs_zero).reshape(nb, SUB_R, SUB_C)
    labs = cluster_ids.reshape(nb, SUB_R, SUB_C)
    stride_arr = jnp.asarray(
        spatial_features_stride).astype(jnp.float32).reshape(1)

    meta_i, meta_f = _fps_call(stride_arr, xs, ys, zs, labs)
    mi_r = meta_i.reshape(nb * 5 * NKP)
    mf_r = meta_f.reshape(nb * 4 * NKP)

    c3d = spatial_features.shape[1]
    c2d = spatial_features_2d.shape[1]
    cout = c3d + c2d
    sf1r = spatial_features.reshape(nb * c3d * H * W)
    sf2r = spatial_features_2d.reshape(nb * c2d * H * W)
    flat = _sc_call(sf1r, sf2r, mi_r, mf_r, nb, cout)
    return flat.reshape(nb * NCLU, cout)
